# baseline (speedup 1.0000x reference)
import functools

import jax
import jax.numpy as jnp
from jax import lax
from jax.experimental import pallas as pl
from jax.experimental.pallas import tpu as pltpu
from jax.experimental.pallas import tpu_sc as plsc

N = 16384
M = 2048
NT = 512
NUM_TILES = N // NT
NUM_ITER = 4
BASE_ALPHA = 0.1

NUM_WORKERS = 32
CHUNK = N // NUM_WORKERS
GRP = 16


def _nn_body(refined_ref, pb2_ref, b2_ref, idx_ref, md_ref, mx_ref):
    pb2 = pb2_ref[0]
    b2 = b2_ref[0]

    def tile(t, acc):
        s = pl.ds(t * NT, NT)
        rall = refined_ref[0, :, s]
        rx = rall[0:1]
        ry = rall[1:2]
        rz = rall[2:3]
        a2 = rx * rx + ry * ry + rz * rz
        dot2 = lax.dot_general(pb2, rall.astype(jnp.bfloat16),
                               (((1,), (0,)), ((), ())),
                               preferred_element_type=jnp.float32)
        dist = jnp.sqrt(jnp.maximum((a2 + b2) - dot2, 1e-12))
        m = jnp.min(dist, axis=0)
        idx = jnp.argmin(dist, axis=0)
        md_ref[0, 0, s] = m
        idx_ref[0, 0, s] = idx
        return jnp.maximum(acc, jnp.max(m))

    maxdist = lax.fori_loop(0, NUM_TILES, tile, jnp.float32(-jnp.inf))
    mx_ref[0, 0, :] = jnp.full((128,), maxdist, jnp.float32)


def _nn_search(refined_t, pb2, b2):
    return pl.pallas_call(
        _nn_body,
        grid=(1,),
        in_specs=[
            pl.BlockSpec((1, 3, N), lambda b: (b, 0, 0)),
            pl.BlockSpec((1, M, 3), lambda b: (b, 0, 0)),
            pl.BlockSpec((1, M, 1), lambda b: (b, 0, 0)),
        ],
        out_specs=[
            pl.BlockSpec((1, 1, N), lambda b: (b, 0, 0)),
            pl.BlockSpec((1, 1, N), lambda b: (b, 0, 0)),
            pl.BlockSpec((1, 1, 128), lambda b: (b, 0, 0)),
        ],
        out_shape=[
            jax.ShapeDtypeStruct((1, 1, N), jnp.int32),
            jax.ShapeDtypeStruct((1, 1, N), jnp.float32),
            jax.ShapeDtypeStruct((1, 1, 128), jnp.float32),
        ],
    )(refined_t, pb2, b2)


def _sc_update_body(refined_hbm, partial_hbm, idx_hbm, md_hbm, mx_hbm,
                    out_hbm, ptab, rxv, ryv, rzv, idxv, mdv, mxv):
    wid = lax.axis_index("s") * 2 + lax.axis_index("c")
    qbase = wid * CHUNK

    pltpu.sync_copy(partial_hbm, ptab)
    pltpu.sync_copy(refined_hbm.at[pl.ds(qbase, CHUNK)], rxv)
    pltpu.sync_copy(refined_hbm.at[pl.ds(qbase + N, CHUNK)], ryv)
    pltpu.sync_copy(refined_hbm.at[pl.ds(qbase + 2 * N, CHUNK)], rzv)
    pltpu.sync_copy(idx_hbm.at[pl.ds(qbase, CHUNK)], idxv)
    pltpu.sync_copy(md_hbm.at[pl.ds(qbase, CHUNK)], mdv)
    pltpu.sync_copy(mx_hbm.at[pl.ds(0, GRP)], mxv)

    denom = mxv[...] + 1e-6

    def step(i, carry):
        s = pl.ds(i * GRP, GRP)
        nn3 = idxv[s] * 3
        nx = plsc.load_gather(ptab, [nn3])
        ny = plsc.load_gather(ptab, [nn3 + 1])
        nz = plsc.load_gather(ptab, [nn3 + 2])
        alpha = BASE_ALPHA * (2.0 - mdv[s] / denom)
        rx, ry, rz = rxv[s], ryv[s], rzv[s]
        rxv[s] = rx + alpha * (nx - rx)
        ryv[s] = ry + alpha * (ny - ry)
        rzv[s] = rz + alpha * (nz - rz)
        return carry

    lax.fori_loop(0, CHUNK // GRP, step, 0)

    pltpu.sync_copy(rxv, out_hbm.at[pl.ds(qbase, CHUNK)])
    pltpu.sync_copy(ryv, out_hbm.at[pl.ds(qbase + N, CHUNK)])
    pltpu.sync_copy(rzv, out_hbm.at[pl.ds(qbase + 2 * N, CHUNK)])


def _sc_update(refined_t, partial, idx, md, mx):
    fn = functools.partial(
        pl.kernel,
        out_type=jax.ShapeDtypeStruct((3 * N,), jnp.float32),
        mesh=plsc.VectorSubcoreMesh(core_axis_name="c", subcore_axis_name="s"),
        compiler_params=pltpu.CompilerParams(needs_layout_passes=False),
        scratch_types=[
            pltpu.VMEM((3 * M,), jnp.float32),
            pltpu.VMEM((CHUNK,), jnp.float32),
            pltpu.VMEM((CHUNK,), jnp.float32),
            pltpu.VMEM((CHUNK,), jnp.float32),
            pltpu.VMEM((CHUNK,), jnp.int32),
            pltpu.VMEM((CHUNK,), jnp.float32),
            pltpu.VMEM((GRP,), jnp.float32),
        ],
    )(_sc_update_body)
    out = fn(refined_t.reshape(-1), partial.reshape(-1), idx.reshape(-1),
             md.reshape(-1), mx.reshape(-1))
    return out.reshape(1, 3, N)


@jax.jit
def kernel(pred, partial):
    B = pred.shape[0]
    pred_t = jnp.swapaxes(pred, 1, 2)
    refined = [pred_t[b:b + 1] for b in range(B)]
    parts = [partial[b:b + 1] for b in range(B)]
    pb2s = [(2.0 * p).astype(jnp.bfloat16) for p in parts]
    b2s = [jnp.sum(p * p, axis=-1, keepdims=True) for p in parts]
    for _ in range(NUM_ITER):
        for b in range(B):
            idx, md, mx = _nn_search(refined[b], pb2s[b], b2s[b])
            refined[b] = _sc_update(refined[b], parts[b], idx, md, mx)
    return jnp.swapaxes(jnp.concatenate(refined, axis=0), 1, 2)


# speedup vs baseline: 1.3320x; 1.3320x over previous
import functools

import jax
import jax.numpy as jnp
from jax import lax
from jax.experimental import pallas as pl
from jax.experimental.pallas import tpu as pltpu
from jax.experimental.pallas import tpu_sc as plsc

N = 16384
M = 2048
NT = 512
NUM_TILES = N // NT
NUM_ITER = 4
BASE_ALPHA = 0.1

NUM_WORKERS = 32
CHUNK = N // NUM_WORKERS
GRP = 16


def _nn_body(refined_ref, pb2_ref, b2_ref, idx_ref, md_ref, mx_ref):
    pb2 = pb2_ref[0]
    b2 = b2_ref[0]
    iota = lax.broadcasted_iota(jnp.int32, (M, NT), 0)

    def tile(t, acc):
        s = pl.ds(t * NT, NT)
        rall = refined_ref[0, :, s]
        rx = rall[0:1]
        ry = rall[1:2]
        rz = rall[2:3]
        a2 = rx * rx + ry * ry + rz * rz
        dot2 = lax.dot_general(pb2, rall.astype(jnp.bfloat16),
                               (((1,), (0,)), ((), ())),
                               preferred_element_type=jnp.float32)
        d2 = jnp.maximum((a2 + b2) - dot2, 1e-12)
        m = jnp.min(d2, axis=0)
        idx = jnp.min(jnp.where(d2 <= m[None, :], iota, M), axis=0)
        md_ref[0, 0, s] = jnp.sqrt(m)
        idx_ref[0, 0, s] = idx
        return jnp.maximum(acc, jnp.max(m))

    maxd2 = lax.fori_loop(0, NUM_TILES, tile, jnp.float32(-jnp.inf))
    mx_ref[0, 0, :] = jnp.full((128,), jnp.sqrt(maxd2), jnp.float32)


def _nn_search(refined_t, pb2, b2):
    return pl.pallas_call(
        _nn_body,
        grid=(1,),
        in_specs=[
            pl.BlockSpec((1, 3, N), lambda b: (b, 0, 0)),
            pl.BlockSpec((1, M, 3), lambda b: (b, 0, 0)),
            pl.BlockSpec((1, M, 1), lambda b: (b, 0, 0)),
        ],
        out_specs=[
            pl.BlockSpec((1, 1, N), lambda b: (b, 0, 0)),
            pl.BlockSpec((1, 1, N), lambda b: (b, 0, 0)),
            pl.BlockSpec((1, 1, 128), lambda b: (b, 0, 0)),
        ],
        out_shape=[
            jax.ShapeDtypeStruct((1, 1, N), jnp.int32),
            jax.ShapeDtypeStruct((1, 1, N), jnp.float32),
            jax.ShapeDtypeStruct((1, 1, 128), jnp.float32),
        ],
    )(refined_t, pb2, b2)


def _sc_update_body(refined_hbm, partial_hbm, idx_hbm, md_hbm, mx_hbm,
                    out_hbm, ptab, rxv, ryv, rzv, idxv, mdv, mxv):
    wid = lax.axis_index("s") * 2 + lax.axis_index("c")
    qbase = wid * CHUNK

    pltpu.sync_copy(partial_hbm, ptab)
    pltpu.sync_copy(refined_hbm.at[pl.ds(qbase, CHUNK)], rxv)
    pltpu.sync_copy(refined_hbm.at[pl.ds(qbase + N, CHUNK)], ryv)
    pltpu.sync_copy(refined_hbm.at[pl.ds(qbase + 2 * N, CHUNK)], rzv)
    pltpu.sync_copy(idx_hbm.at[pl.ds(qbase, CHUNK)], idxv)
    pltpu.sync_copy(md_hbm.at[pl.ds(qbase, CHUNK)], mdv)
    pltpu.sync_copy(mx_hbm.at[pl.ds(0, GRP)], mxv)

    denom = mxv[...] + 1e-6

    def step(i, carry):
        s = pl.ds(i * GRP, GRP)
        nn3 = idxv[s] * 3
        nx = plsc.load_gather(ptab, [nn3])
        ny = plsc.load_gather(ptab, [nn3 + 1])
        nz = plsc.load_gather(ptab, [nn3 + 2])
        alpha = BASE_ALPHA * (2.0 - mdv[s] / denom)
        rx, ry, rz = rxv[s], ryv[s], rzv[s]
        rxv[s] = rx + alpha * (nx - rx)
        ryv[s] = ry + alpha * (ny - ry)
        rzv[s] = rz + alpha * (nz - rz)
        return carry

    lax.fori_loop(0, CHUNK // GRP, step, 0)

    pltpu.sync_copy(rxv, out_hbm.at[pl.ds(qbase, CHUNK)])
    pltpu.sync_copy(ryv, out_hbm.at[pl.ds(qbase + N, CHUNK)])
    pltpu.sync_copy(rzv, out_hbm.at[pl.ds(qbase + 2 * N, CHUNK)])


def _sc_update(refined_t, partial, idx, md, mx):
    fn = functools.partial(
        pl.kernel,
        out_type=jax.ShapeDtypeStruct((3 * N,), jnp.float32),
        mesh=plsc.VectorSubcoreMesh(core_axis_name="c", subcore_axis_name="s"),
        compiler_params=pltpu.CompilerParams(needs_layout_passes=False),
        scratch_types=[
            pltpu.VMEM((3 * M,), jnp.float32),
            pltpu.VMEM((CHUNK,), jnp.float32),
            pltpu.VMEM((CHUNK,), jnp.float32),
            pltpu.VMEM((CHUNK,), jnp.float32),
            pltpu.VMEM((CHUNK,), jnp.int32),
            pltpu.VMEM((CHUNK,), jnp.float32),
            pltpu.VMEM((GRP,), jnp.float32),
        ],
    )(_sc_update_body)
    out = fn(refined_t.reshape(-1), partial.reshape(-1), idx.reshape(-1),
             md.reshape(-1), mx.reshape(-1))
    return out.reshape(1, 3, N)


@jax.jit
def kernel(pred, partial):
    B = pred.shape[0]
    pred_t = jnp.swapaxes(pred, 1, 2)
    refined = [pred_t[b:b + 1] for b in range(B)]
    parts = [partial[b:b + 1] for b in range(B)]
    pb2s = [(2.0 * p).astype(jnp.bfloat16) for p in parts]
    b2s = [jnp.sum(p * p, axis=-1, keepdims=True) for p in parts]
    for _ in range(NUM_ITER):
        for b in range(B):
            idx, md, mx = _nn_search(refined[b], pb2s[b], b2s[b])
            refined[b] = _sc_update(refined[b], parts[b], idx, md, mx)
    return jnp.swapaxes(jnp.concatenate(refined, axis=0), 1, 2)


# speedup vs baseline: 1.5902x; 1.1938x over previous
import functools

import jax
import jax.numpy as jnp
from jax import lax
from jax.experimental import pallas as pl
from jax.experimental.pallas import tpu as pltpu
from jax.experimental.pallas import tpu_sc as plsc

N = 16384
M = 2048
NT = 512
NUM_TILES = N // NT
NUM_ITER = 4
BASE_ALPHA = 0.1

NUM_WORKERS = 32
CHUNK = N // NUM_WORKERS
GRP = 16


def _nn_body(refined_ref, pb2_ref, b2_ref, idx_ref, md_ref, mx_ref):
    pb2 = pb2_ref[0]
    b2 = b2_ref[0]
    iota = lax.broadcasted_iota(jnp.int32, (M, NT), 0)

    def tile(t, acc):
        s = pl.ds(t * NT, NT)
        rall = refined_ref[0, :, s]
        rx = rall[0:1]
        ry = rall[1:2]
        rz = rall[2:3]
        a2 = rx * rx + ry * ry + rz * rz
        dot2 = lax.dot_general(pb2, rall.astype(jnp.bfloat16),
                               (((1,), (0,)), ((), ())),
                               preferred_element_type=jnp.float32)
        d2 = jnp.maximum((a2 + b2) - dot2, 1e-12)
        m = jnp.min(d2, axis=0)
        idx = jnp.argmin(d2, axis=0)
        md_ref[0, 0, s] = jnp.sqrt(m)
        idx_ref[0, 0, s] = idx
        return jnp.maximum(acc, jnp.max(m))

    maxd2 = lax.fori_loop(0, NUM_TILES, tile, jnp.float32(-jnp.inf))
    mx_ref[0, 0, :] = jnp.full((128,), jnp.sqrt(maxd2), jnp.float32)


def _nn_search(refined_t, pb2, b2):
    return pl.pallas_call(
        _nn_body,
        grid=(1,),
        in_specs=[
            pl.BlockSpec((1, 3, N), lambda b: (b, 0, 0)),
            pl.BlockSpec((1, M, 3), lambda b: (b, 0, 0)),
            pl.BlockSpec((1, M, 1), lambda b: (b, 0, 0)),
        ],
        out_specs=[
            pl.BlockSpec((1, 1, N), lambda b: (b, 0, 0)),
            pl.BlockSpec((1, 1, N), lambda b: (b, 0, 0)),
            pl.BlockSpec((1, 1, 128), lambda b: (b, 0, 0)),
        ],
        out_shape=[
            jax.ShapeDtypeStruct((1, 1, N), jnp.int32),
            jax.ShapeDtypeStruct((1, 1, N), jnp.float32),
            jax.ShapeDtypeStruct((1, 1, 128), jnp.float32),
        ],
    )(refined_t, pb2, b2)


def _sc_update_body(refined_hbm, partial_hbm, idx_hbm, md_hbm, mx_hbm,
                    out_hbm, ptab, rxv, ryv, rzv, idxv, mdv, mxv):
    wid = lax.axis_index("s") * 2 + lax.axis_index("c")
    qbase = wid * CHUNK

    pltpu.sync_copy(partial_hbm, ptab)
    pltpu.sync_copy(refined_hbm.at[pl.ds(qbase, CHUNK)], rxv)
    pltpu.sync_copy(refined_hbm.at[pl.ds(qbase + N, CHUNK)], ryv)
    pltpu.sync_copy(refined_hbm.at[pl.ds(qbase + 2 * N, CHUNK)], rzv)
    pltpu.sync_copy(idx_hbm.at[pl.ds(qbase, CHUNK)], idxv)
    pltpu.sync_copy(md_hbm.at[pl.ds(qbase, CHUNK)], mdv)
    pltpu.sync_copy(mx_hbm.at[pl.ds(0, GRP)], mxv)

    denom = mxv[...] + 1e-6

    def step(i, carry):
        s = pl.ds(i * GRP, GRP)
        nn3 = idxv[s] * 3
        nx = plsc.load_gather(ptab, [nn3])
        ny = plsc.load_gather(ptab, [nn3 + 1])
        nz = plsc.load_gather(ptab, [nn3 + 2])
        alpha = BASE_ALPHA * (2.0 - mdv[s] / denom)
        rx, ry, rz = rxv[s], ryv[s], rzv[s]
        rxv[s] = rx + alpha * (nx - rx)
        ryv[s] = ry + alpha * (ny - ry)
        rzv[s] = rz + alpha * (nz - rz)
        return carry

    lax.fori_loop(0, CHUNK // GRP, step, 0)

    pltpu.sync_copy(rxv, out_hbm.at[pl.ds(qbase, CHUNK)])
    pltpu.sync_copy(ryv, out_hbm.at[pl.ds(qbase + N, CHUNK)])
    pltpu.sync_copy(rzv, out_hbm.at[pl.ds(qbase + 2 * N, CHUNK)])


def _sc_update(refined_t, partial, idx, md, mx):
    fn = functools.partial(
        pl.kernel,
        out_type=jax.ShapeDtypeStruct((3 * N,), jnp.float32),
        mesh=plsc.VectorSubcoreMesh(core_axis_name="c", subcore_axis_name="s"),
        compiler_params=pltpu.CompilerParams(needs_layout_passes=False),
        scratch_types=[
            pltpu.VMEM((3 * M,), jnp.float32),
            pltpu.VMEM((CHUNK,), jnp.float32),
            pltpu.VMEM((CHUNK,), jnp.float32),
            pltpu.VMEM((CHUNK,), jnp.float32),
            pltpu.VMEM((CHUNK,), jnp.int32),
            pltpu.VMEM((CHUNK,), jnp.float32),
            pltpu.VMEM((GRP,), jnp.float32),
        ],
    )(_sc_update_body)
    out = fn(refined_t.reshape(-1), partial.reshape(-1), idx.reshape(-1),
             md.reshape(-1), mx.reshape(-1))
    return out.reshape(1, 3, N)


@jax.jit
def kernel(pred, partial):
    B = pred.shape[0]
    pred_t = jnp.swapaxes(pred, 1, 2)
    refined = [pred_t[b:b + 1] for b in range(B)]
    parts = [partial[b:b + 1] for b in range(B)]
    pb2s = [(2.0 * p).astype(jnp.bfloat16) for p in parts]
    b2s = [jnp.sum(p * p, axis=-1, keepdims=True) for p in parts]
    for _ in range(NUM_ITER):
        for b in range(B):
            idx, md, mx = _nn_search(refined[b], pb2s[b], b2s[b])
            refined[b] = _sc_update(refined[b], parts[b], idx, md, mx)
    return jnp.swapaxes(jnp.concatenate(refined, axis=0), 1, 2)


# speedup vs baseline: 1.5989x; 1.0055x over previous
import functools

import jax
import jax.numpy as jnp
from jax import lax
from jax.experimental import pallas as pl
from jax.experimental.pallas import tpu as pltpu
from jax.experimental.pallas import tpu_sc as plsc

N = 16384
M = 2048
NT = 512
NUM_TILES = N // NT
NUM_ITER = 4
BASE_ALPHA = 0.1

NUM_WORKERS = 32
CHUNK = N // NUM_WORKERS
GRP = 16


def _nn_body(refined_ref, pb2_ref, b2_ref, idx_ref, md_ref, mx_ref):
    pb2 = pb2_ref[0]
    b2 = b2_ref[0]

    def tile(t, acc):
        s = pl.ds(t * NT, NT)
        rall = refined_ref[0, :, s]
        rx = rall[0:1]
        ry = rall[1:2]
        rz = rall[2:3]
        a2 = rx * rx + ry * ry + rz * rz
        dot2 = lax.dot_general(pb2, rall.astype(jnp.bfloat16),
                               (((1,), (0,)), ((), ())),
                               preferred_element_type=jnp.float32)
        d2 = jnp.maximum((a2 + b2) - dot2, 1e-12)
        m = jnp.min(d2, axis=0)
        idx = jnp.argmin(d2, axis=0)
        md_ref[0, 0, s] = jnp.sqrt(m)
        idx_ref[0, 0, s] = idx
        return jnp.maximum(acc, jnp.max(m))

    maxd2 = lax.fori_loop(0, NUM_TILES, tile, jnp.float32(-jnp.inf))
    mx_ref[0, 0, :] = jnp.full((128,), jnp.sqrt(maxd2), jnp.float32)


def _nn_search(refined_t, pb2, b2):
    return pl.pallas_call(
        _nn_body,
        grid=(1,),
        in_specs=[
            pl.BlockSpec((1, 3, N), lambda b: (b, 0, 0)),
            pl.BlockSpec((1, M, 3), lambda b: (b, 0, 0)),
            pl.BlockSpec((1, M, 1), lambda b: (b, 0, 0)),
        ],
        out_specs=[
            pl.BlockSpec((1, 1, N), lambda b: (b, 0, 0)),
            pl.BlockSpec((1, 1, N), lambda b: (b, 0, 0)),
            pl.BlockSpec((1, 1, 128), lambda b: (b, 0, 0)),
        ],
        out_shape=[
            jax.ShapeDtypeStruct((1, 1, N), jnp.int32),
            jax.ShapeDtypeStruct((1, 1, N), jnp.float32),
            jax.ShapeDtypeStruct((1, 1, 128), jnp.float32),
        ],
    )(refined_t, pb2, b2)


def _sc_update_body(refined_hbm, partial_hbm, idx_hbm, md_hbm, mx_hbm,
                    out_hbm, ptab, rxv, ryv, rzv, idxv, mdv, mxv, sem):
    wid = lax.axis_index("s") * 2 + lax.axis_index("c")
    qbase = wid * CHUNK

    copies = [
        pltpu.async_copy(partial_hbm, ptab, sem),
        pltpu.async_copy(refined_hbm.at[pl.ds(qbase, CHUNK)], rxv, sem),
        pltpu.async_copy(refined_hbm.at[pl.ds(qbase + N, CHUNK)], ryv, sem),
        pltpu.async_copy(refined_hbm.at[pl.ds(qbase + 2 * N, CHUNK)], rzv, sem),
        pltpu.async_copy(idx_hbm.at[pl.ds(qbase, CHUNK)], idxv, sem),
        pltpu.async_copy(md_hbm.at[pl.ds(qbase, CHUNK)], mdv, sem),
        pltpu.async_copy(mx_hbm.at[pl.ds(0, GRP)], mxv, sem),
    ]
    for c in copies:
        c.wait()

    denom = mxv[...] + 1e-6

    def step(i, carry):
        s = pl.ds(i * GRP, GRP)
        nn3 = idxv[s] * 3
        nx = plsc.load_gather(ptab, [nn3])
        ny = plsc.load_gather(ptab, [nn3 + 1])
        nz = plsc.load_gather(ptab, [nn3 + 2])
        alpha = BASE_ALPHA * (2.0 - mdv[s] / denom)
        rx, ry, rz = rxv[s], ryv[s], rzv[s]
        rxv[s] = rx + alpha * (nx - rx)
        ryv[s] = ry + alpha * (ny - ry)
        rzv[s] = rz + alpha * (nz - rz)
        return carry

    lax.fori_loop(0, CHUNK // GRP, step, 0)

    stores = [
        pltpu.async_copy(rxv, out_hbm.at[pl.ds(qbase, CHUNK)], sem),
        pltpu.async_copy(ryv, out_hbm.at[pl.ds(qbase + N, CHUNK)], sem),
        pltpu.async_copy(rzv, out_hbm.at[pl.ds(qbase + 2 * N, CHUNK)], sem),
    ]
    for c in stores:
        c.wait()


def _sc_update(refined_t, partial, idx, md, mx):
    fn = functools.partial(
        pl.kernel,
        out_type=jax.ShapeDtypeStruct((3 * N,), jnp.float32),
        mesh=plsc.VectorSubcoreMesh(core_axis_name="c", subcore_axis_name="s"),
        compiler_params=pltpu.CompilerParams(needs_layout_passes=False),
        scratch_types=[
            pltpu.VMEM((3 * M,), jnp.float32),
            pltpu.VMEM((CHUNK,), jnp.float32),
            pltpu.VMEM((CHUNK,), jnp.float32),
            pltpu.VMEM((CHUNK,), jnp.float32),
            pltpu.VMEM((CHUNK,), jnp.int32),
            pltpu.VMEM((CHUNK,), jnp.float32),
            pltpu.VMEM((GRP,), jnp.float32),
            pltpu.SemaphoreType.DMA,
        ],
    )(_sc_update_body)
    out = fn(refined_t.reshape(-1), partial.reshape(-1), idx.reshape(-1),
             md.reshape(-1), mx.reshape(-1))
    return out.reshape(1, 3, N)


@jax.jit
def kernel(pred, partial):
    B = pred.shape[0]
    pred_t = jnp.swapaxes(pred, 1, 2)
    refined = [pred_t[b:b + 1] for b in range(B)]
    parts = [partial[b:b + 1] for b in range(B)]
    pb2s = [(2.0 * p).astype(jnp.bfloat16) for p in parts]
    b2s = [jnp.sum(p * p, axis=-1, keepdims=True) for p in parts]
    for _ in range(NUM_ITER):
        for b in range(B):
            idx, md, mx = _nn_search(refined[b], pb2s[b], b2s[b])
            refined[b] = _sc_update(refined[b], parts[b], idx, md, mx)
    return jnp.swapaxes(jnp.concatenate(refined, axis=0), 1, 2)


# speedup vs baseline: 1.6460x; 1.0294x over previous
import functools

import jax
import jax.numpy as jnp
from jax import lax
from jax.experimental import pallas as pl
from jax.experimental.pallas import tpu as pltpu
from jax.experimental.pallas import tpu_sc as plsc

N = 16384
M = 2048
NT = 512
NUM_TILES = N // NT
NUM_ITER = 4
BASE_ALPHA = 0.1

NUM_WORKERS = 32
CHUNK = 2 * N // NUM_WORKERS
GRP = 16


def _nn_body(refined_ref, pb2_ref, b2_ref, idx_ref, md_ref, mx_ref):
    pb2 = pb2_ref[0]
    b2 = b2_ref[0]

    def tile(t, acc):
        s = pl.ds(t * NT, NT)
        rall = refined_ref[0, :, s]
        rx = rall[0:1]
        ry = rall[1:2]
        rz = rall[2:3]
        a2 = rx * rx + ry * ry + rz * rz
        dot2 = lax.dot_general(pb2, rall.astype(jnp.bfloat16),
                               (((1,), (0,)), ((), ())),
                               preferred_element_type=jnp.float32)
        d2 = jnp.maximum((a2 + b2) - dot2, 1e-12)
        m = jnp.min(d2, axis=0)
        idx = jnp.argmin(d2, axis=0)
        md_ref[0, 0, s] = jnp.sqrt(m)
        idx_ref[0, 0, s] = idx
        return jnp.maximum(acc, jnp.max(m))

    maxd2 = lax.fori_loop(0, NUM_TILES, tile, jnp.float32(-jnp.inf))
    mx_ref[0, 0, :] = jnp.full((128,), jnp.sqrt(maxd2), jnp.float32)


def _nn_search(refined_t, pb2, b2):
    B = refined_t.shape[0]
    return pl.pallas_call(
        _nn_body,
        grid=(B,),
        in_specs=[
            pl.BlockSpec((1, 3, N), lambda b: (b, 0, 0)),
            pl.BlockSpec((1, M, 3), lambda b: (b, 0, 0)),
            pl.BlockSpec((1, M, 1), lambda b: (b, 0, 0)),
        ],
        out_specs=[
            pl.BlockSpec((1, 1, N), lambda b: (b, 0, 0)),
            pl.BlockSpec((1, 1, N), lambda b: (b, 0, 0)),
            pl.BlockSpec((1, 1, 128), lambda b: (b, 0, 0)),
        ],
        out_shape=[
            jax.ShapeDtypeStruct((B, 1, N), jnp.int32),
            jax.ShapeDtypeStruct((B, 1, N), jnp.float32),
            jax.ShapeDtypeStruct((B, 1, 128), jnp.float32),
        ],
    )(refined_t, pb2, b2)


def _sc_update_body(refined_hbm, partial_hbm, idx_hbm, md_hbm, mx_hbm,
                    out_hbm, ptab, rxv, ryv, rzv, idxv, mdv, mxv, sem):
    wid = lax.axis_index("s") * 2 + lax.axis_index("c")
    b = wid // (NUM_WORKERS // 2)
    qbase = wid * CHUNK
    rbase = b * 3 * N + (wid % (NUM_WORKERS // 2)) * CHUNK

    copies = [
        pltpu.async_copy(partial_hbm.at[pl.ds(b * 3 * M, 3 * M)], ptab, sem),
        pltpu.async_copy(refined_hbm.at[pl.ds(rbase, CHUNK)], rxv, sem),
        pltpu.async_copy(refined_hbm.at[pl.ds(rbase + N, CHUNK)], ryv, sem),
        pltpu.async_copy(refined_hbm.at[pl.ds(rbase + 2 * N, CHUNK)], rzv, sem),
        pltpu.async_copy(idx_hbm.at[pl.ds(qbase, CHUNK)], idxv, sem),
        pltpu.async_copy(md_hbm.at[pl.ds(qbase, CHUNK)], mdv, sem),
        pltpu.async_copy(mx_hbm.at[pl.ds(b * 128, GRP)], mxv, sem),
    ]
    for c in copies:
        c.wait()

    denom = mxv[...] + 1e-6

    def step(i, carry):
        s = pl.ds(i * GRP, GRP)
        nn3 = idxv[s] * 3
        nx = plsc.load_gather(ptab, [nn3])
        ny = plsc.load_gather(ptab, [nn3 + 1])
        nz = plsc.load_gather(ptab, [nn3 + 2])
        alpha = BASE_ALPHA * (2.0 - mdv[s] / denom)
        rx, ry, rz = rxv[s], ryv[s], rzv[s]
        rxv[s] = rx + alpha * (nx - rx)
        ryv[s] = ry + alpha * (ny - ry)
        rzv[s] = rz + alpha * (nz - rz)
        return carry

    lax.fori_loop(0, CHUNK // GRP, step, 0)

    stores = [
        pltpu.async_copy(rxv, out_hbm.at[pl.ds(rbase, CHUNK)], sem),
        pltpu.async_copy(ryv, out_hbm.at[pl.ds(rbase + N, CHUNK)], sem),
        pltpu.async_copy(rzv, out_hbm.at[pl.ds(rbase + 2 * N, CHUNK)], sem),
    ]
    for c in stores:
        c.wait()


def _sc_update(refined_t, partial, idx, md, mx):
    B = refined_t.shape[0]
    fn = functools.partial(
        pl.kernel,
        out_type=jax.ShapeDtypeStruct((B * 3 * N,), jnp.float32),
        mesh=plsc.VectorSubcoreMesh(core_axis_name="c", subcore_axis_name="s"),
        compiler_params=pltpu.CompilerParams(needs_layout_passes=False),
        scratch_types=[
            pltpu.VMEM((3 * M,), jnp.float32),
            pltpu.VMEM((CHUNK,), jnp.float32),
            pltpu.VMEM((CHUNK,), jnp.float32),
            pltpu.VMEM((CHUNK,), jnp.float32),
            pltpu.VMEM((CHUNK,), jnp.int32),
            pltpu.VMEM((CHUNK,), jnp.float32),
            pltpu.VMEM((GRP,), jnp.float32),
            pltpu.SemaphoreType.DMA,
        ],
    )(_sc_update_body)
    out = fn(refined_t.reshape(-1), partial.reshape(-1), idx.reshape(-1),
             md.reshape(-1), mx.reshape(-1))
    return out.reshape(B, 3, N)


@jax.jit
def kernel(pred, partial):
    B = pred.shape[0]
    refined = jnp.swapaxes(pred, 1, 2)
    pb2 = (2.0 * partial).astype(jnp.bfloat16)
    b2 = jnp.sum(partial * partial, axis=-1, keepdims=True)
    for _ in range(NUM_ITER):
        idx, md, mx = _nn_search(refined, pb2, b2)
        refined = _sc_update(refined, partial, idx, md, mx)
    return jnp.swapaxes(refined, 1, 2)
